# baseline (speedup 1.0000x reference)
import functools

import jax
import jax.numpy as jnp
from jax import lax
from jax.experimental import pallas as pl
from jax.experimental.pallas import tpu as pltpu
from jax.experimental.pallas import tpu_sc as plsc

NC = 2
NS = 16
NW = NC * NS

BATCH = 16384
D = 32
N_ROWS = 1000000
B_PER_W = BATCH // NW
CHUNK = 128
N_CHUNK = B_PER_W // CHUNK


def _detile_body(*refs):
    u_hbm, i_hbm = refs[0], refs[1]
    u_out = refs[2:2 + D]
    i_out = refs[2 + D:2 + 2 * D]
    sem = refs[-1]
    copies = []
    for d in range(D):
        copies.append(pltpu.async_copy(u_hbm.at[d], u_out[d], sem))
        copies.append(pltpu.async_copy(i_hbm.at[d], i_out[d], sem))
    for c in copies:
        c.wait()


def _tc_detile(ut_t, it_t):
    plane = jax.ShapeDtypeStruct((N_ROWS,), jnp.float32)
    outs = pl.pallas_call(
        _detile_body,
        in_specs=[pl.BlockSpec(memory_space=pl.ANY),
                  pl.BlockSpec(memory_space=pl.ANY)],
        out_specs=[pl.BlockSpec(memory_space=pl.ANY)] * (2 * D),
        out_shape=[plane] * (2 * D),
        scratch_shapes=[pltpu.SemaphoreType.DMA],
    )(ut_t, it_t)
    return outs[:D], outs[D:]


def _gather_body(*refs):
    u_planes = refs[0:D]
    i_planes = refs[D:2 * D]
    ui_hbm, ii_hbm = refs[2 * D], refs[2 * D + 1]
    u_flat, v_flat = refs[2 * D + 2], refs[2 * D + 3]
    uidx_v, iidx_v, urows_v, irows_v, sem_i, sem_g, sem_o = refs[2 * D + 4:]

    wid = lax.axis_index("s") * NC + lax.axis_index("c")
    base = wid * B_PER_W
    pltpu.async_copy(ui_hbm.at[pl.ds(base, B_PER_W)], uidx_v, sem_i)
    pltpu.async_copy(ii_hbm.at[pl.ds(base, B_PER_W)], iidx_v, sem_i).wait()
    pltpu.make_async_copy(ui_hbm.at[pl.ds(base, B_PER_W)], uidx_v, sem_i).wait()

    gathers = []
    for d in range(D):
        for j in range(N_CHUNK):
            gathers.append(pltpu.async_copy(
                u_planes[d].at[uidx_v.at[pl.ds(j * CHUNK, CHUNK)]],
                urows_v.at[pl.ds(d * B_PER_W + j * CHUNK, CHUNK)], sem_g))
            gathers.append(pltpu.async_copy(
                i_planes[d].at[iidx_v.at[pl.ds(j * CHUNK, CHUNK)]],
                irows_v.at[pl.ds(d * B_PER_W + j * CHUNK, CHUNK)], sem_g))
    for g in gathers:
        g.wait()

    writes = []
    for d in range(D):
        writes.append(pltpu.async_copy(
            urows_v.at[pl.ds(d * B_PER_W, B_PER_W)],
            u_flat.at[pl.ds(d * BATCH + base, B_PER_W)], sem_o))
        writes.append(pltpu.async_copy(
            irows_v.at[pl.ds(d * B_PER_W, B_PER_W)],
            v_flat.at[pl.ds(d * BATCH + base, B_PER_W)], sem_o))
    for w in writes:
        w.wait()


def _sc_gather(u_planes, i_planes, user_idx, item_idx):
    mesh = plsc.VectorSubcoreMesh(core_axis_name="c", subcore_axis_name="s")
    out_t = jax.ShapeDtypeStruct((D * BATCH,), jnp.float32)
    k = pl.kernel(
        _gather_body,
        out_type=[out_t, out_t],
        mesh=mesh,
        compiler_params=pltpu.CompilerParams(use_tc_tiling_on_sc=False),
        scratch_types=[
            pltpu.VMEM((B_PER_W,), jnp.int32),
            pltpu.VMEM((B_PER_W,), jnp.int32),
            pltpu.VMEM((D * B_PER_W,), jnp.float32),
            pltpu.VMEM((D * B_PER_W,), jnp.float32),
            pltpu.SemaphoreType.DMA,
            pltpu.SemaphoreType.DMA,
            pltpu.SemaphoreType.DMA,
        ],
    )
    return k(*u_planes, *i_planes, user_idx, item_idx)


BLK = 2048


def _mlp_body(u_ref, v_ref, w1u_ref, w1v_ref, b1_ref, w2_ref, b2_ref,
              wo_ref, bo_ref, o_ref):
    h = w1u_ref[...] @ u_ref[...] + w1v_ref[...] @ v_ref[...] + b1_ref[...]
    h = jnp.maximum(h, 0.0)
    h = jnp.maximum(w2_ref[...] @ h + b2_ref[...], 0.0)
    o_ref[...] = wo_ref[...] @ h + bo_ref[...]


def _tc_mlp(u_t, v_t, W1, b1, W2, b2, Wo, bo):
    w1ut, w1vt = W1[:D].T, W1[D:].T
    grid = (BATCH // BLK,)
    full = lambda shape: pl.BlockSpec(shape, lambda i: (0, 0))
    out = pl.pallas_call(
        _mlp_body,
        grid=grid,
        in_specs=[
            pl.BlockSpec((D, BLK), lambda i: (0, i)),
            pl.BlockSpec((D, BLK), lambda i: (0, i)),
            full((64, D)),
            full((64, D)),
            full((64, 1)),
            full((32, 64)),
            full((32, 1)),
            full((1, 32)),
            full((1, 1)),
        ],
        out_specs=pl.BlockSpec((1, BLK), lambda i: (0, i)),
        out_shape=jax.ShapeDtypeStruct((1, BATCH), jnp.float32),
    )(u_t, v_t, w1ut, w1vt, b1.reshape(64, 1), W2.T, b2.reshape(32, 1),
      Wo.T, bo.reshape(1, 1))
    return out.reshape(BATCH)


def kernel(user_indices, item_indices, user_table, item_table,
           W1, b1, W2, b2, Wo, bo):
    u_planes, i_planes = _tc_detile(user_table.T, item_table.T)
    u_flat, v_flat = _sc_gather(u_planes, i_planes, user_indices, item_indices)
    u_t = u_flat.reshape(D, BATCH)
    v_t = v_flat.reshape(D, BATCH)
    return _tc_mlp(u_t, v_t, W1, b1, W2, b2, Wo, bo)


# speedup vs baseline: 16.9005x; 16.9005x over previous
import functools

import jax
import jax.numpy as jnp
from jax import lax
from jax.experimental import pallas as pl
from jax.experimental.pallas import tpu as pltpu
from jax.experimental.pallas import tpu_sc as plsc

NC = 2
NS = 16
NW = NC * NS

BATCH = 16384
D = 32
N_ROWS = 1000000
PW = 2048
P_GRID = (N_ROWS + PW - 1) // PW
QROWS = P_GRID * (PW // 4)
B_PER_W = BATCH // NW
CHUNK = 128
N_CHUNK = B_PER_W // CHUNK
IDX_ROWS = BATCH // CHUNK

SUB = PW // 4


def _pack_body(x0, x1, x2, x3, y0, y1, y2, y3, eye_ref, uo_ref, io_ref):
    e = eye_ref[...]
    xs = jnp.concatenate([x0[...], x1[...], x2[...], x3[...]], axis=0)
    ys = jnp.concatenate([y0[...], y1[...], y2[...], y3[...]], axis=0)
    uo_ref[...] = lax.dot_general(
        xs, e, (((0,), (0,)), ((), ())), preferred_element_type=jnp.float32)
    io_ref[...] = lax.dot_general(
        ys, e, (((0,), (0,)), ((), ())), preferred_element_type=jnp.float32)


def _tc_pack(ut_t, it_t):
    eye = jnp.eye(128, dtype=jnp.float32)
    line_t = jax.ShapeDtypeStruct((QROWS, 128), jnp.float32)
    last = N_ROWS // SUB
    sub = lambda g: pl.BlockSpec(
        (D, SUB), lambda i, g=g: (0, jnp.minimum(4 * i + g, last)))
    uq, iq = pl.pallas_call(
        _pack_body,
        grid=(P_GRID,),
        in_specs=[sub(0), sub(1), sub(2), sub(3),
                  sub(0), sub(1), sub(2), sub(3),
                  pl.BlockSpec((128, 128), lambda i: (0, 0))],
        out_specs=[pl.BlockSpec((SUB, 128), lambda i: (i, 0)),
                   pl.BlockSpec((SUB, 128), lambda i: (i, 0))],
        out_shape=[line_t, line_t],
    )(ut_t, ut_t, ut_t, ut_t, it_t, it_t, it_t, it_t, eye)
    return uq.reshape(QROWS * 4, D), iq.reshape(QROWS * 4, D)


def _gather_body(ut_hbm, it_hbm, ui_hbm, ii_hbm, u_hbm, v_hbm,
                 uidx_v, iidx_v, urows_v, irows_v, sem):
    wid = lax.axis_index("s") * NC + lax.axis_index("c")
    row0 = wid * N_CHUNK
    pltpu.sync_copy(ui_hbm.at[pl.ds(row0, N_CHUNK)], uidx_v)
    pltpu.sync_copy(ii_hbm.at[pl.ds(row0, N_CHUNK)], iidx_v)
    copies = []
    for j in range(N_CHUNK):
        copies.append(pltpu.async_copy(
            ut_hbm.at[uidx_v.at[j]], urows_v.at[pl.ds(j * CHUNK, CHUNK)], sem))
        copies.append(pltpu.async_copy(
            it_hbm.at[iidx_v.at[j]], irows_v.at[pl.ds(j * CHUNK, CHUNK)], sem))
    for c in copies:
        c.wait()
    base = wid * B_PER_W
    pltpu.sync_copy(urows_v, u_hbm.at[pl.ds(base, B_PER_W)])
    pltpu.sync_copy(irows_v, v_hbm.at[pl.ds(base, B_PER_W)])


def _sc_gather(user_table, item_table, user_idx, item_idx):
    mesh = plsc.VectorSubcoreMesh(core_axis_name="c", subcore_axis_name="s")
    rows_t = jax.ShapeDtypeStruct((BATCH, D), jnp.float32)
    k = pl.kernel(
        _gather_body,
        out_type=[rows_t, rows_t],
        mesh=mesh,
        compiler_params=pltpu.CompilerParams(use_tc_tiling_on_sc=False),
        scratch_types=[
            pltpu.VMEM((N_CHUNK, CHUNK), jnp.int32),
            pltpu.VMEM((N_CHUNK, CHUNK), jnp.int32),
            pltpu.VMEM((B_PER_W, D), jnp.float32),
            pltpu.VMEM((B_PER_W, D), jnp.float32),
            pltpu.SemaphoreType.DMA,
        ],
    )
    return k(user_table, item_table,
             user_idx.reshape(IDX_ROWS, CHUNK), item_idx.reshape(IDX_ROWS, CHUNK))


BLK = 2048


def _mlp_body(u_ref, v_ref, w1u_ref, w1v_ref, b1_ref, w2_ref, b2_ref,
              wo_ref, bo_ref, o_ref):
    h = u_ref[...] @ w1u_ref[...] + v_ref[...] @ w1v_ref[...] + b1_ref[...]
    h = jnp.maximum(h, 0.0)
    h = jnp.maximum(h @ w2_ref[...] + b2_ref[...], 0.0)
    o_ref[...] = h @ wo_ref[...] + bo_ref[...]


def _tc_mlp(u, v, W1, b1, W2, b2, Wo, bo):
    w1u, w1v = W1[:D], W1[D:]
    grid = (BATCH // BLK,)
    full = lambda shape: pl.BlockSpec(shape, lambda i: (0, 0))
    out = pl.pallas_call(
        _mlp_body,
        grid=grid,
        in_specs=[
            pl.BlockSpec((BLK, D), lambda i: (i, 0)),
            pl.BlockSpec((BLK, D), lambda i: (i, 0)),
            full((D, 64)),
            full((D, 64)),
            full((1, 64)),
            full((64, 32)),
            full((1, 32)),
            full((32, 1)),
            full((1, 1)),
        ],
        out_specs=pl.BlockSpec((BLK, 1), lambda i: (i, 0)),
        out_shape=jax.ShapeDtypeStruct((BATCH, 1), jnp.float32),
    )(u, v, w1u, w1v, b1.reshape(1, 64), W2, b2.reshape(1, 32),
      Wo, bo.reshape(1, 1))
    return out[:, 0]


def kernel(user_indices, item_indices, user_table, item_table,
           W1, b1, W2, b2, Wo, bo):
    uq, iq = _tc_pack(user_table.T, item_table.T)

    def remap(r):
        i = r // PW
        rem = r % PW
        return 4 * (SUB * i + rem % SUB) + rem // SUB

    uidx = remap(user_indices)
    iidx = remap(item_indices)
    u, v = _sc_gather(uq, iq, uidx, iidx)
    return _tc_mlp(u, v, W1, b1, W2, b2, Wo, bo)


# speedup vs baseline: 23.8177x; 1.4093x over previous
import functools

import jax
import jax.numpy as jnp
from jax import lax
from jax.experimental import pallas as pl
from jax.experimental.pallas import tpu as pltpu
from jax.experimental.pallas import tpu_sc as plsc

NC = 2
NS = 16
NW = NC * NS

BATCH = 16384
D = 32
N_ROWS = 1000000
PW = 4096
P_GRID = (N_ROWS + PW - 1) // PW
QROWS = P_GRID * (PW // 4)
B_PER_W = BATCH // NW
CHUNK = 128
N_CHUNK = B_PER_W // CHUNK
IDX_ROWS = BATCH // CHUNK

SUB = PW // 4


def _pack_body(x0, x1, x2, x3, y0, y1, y2, y3, eye_ref, uo_ref, io_ref):
    e = eye_ref[...]
    xs = jnp.concatenate([x0[...], x1[...], x2[...], x3[...]], axis=0)
    ys = jnp.concatenate([y0[...], y1[...], y2[...], y3[...]], axis=0)
    uo_ref[...] = lax.dot_general(
        xs, e, (((0,), (0,)), ((), ())), preferred_element_type=jnp.float32)
    io_ref[...] = lax.dot_general(
        ys, e, (((0,), (0,)), ((), ())), preferred_element_type=jnp.float32)


def _tc_pack(ut_t, it_t):
    eye = jnp.eye(128, dtype=jnp.float32)
    line_t = jax.ShapeDtypeStruct((QROWS, 128), jnp.float32)
    last = N_ROWS // SUB
    sub = lambda g: pl.BlockSpec(
        (D, SUB), lambda i, g=g: (0, jnp.minimum(4 * i + g, last)))
    uq, iq = pl.pallas_call(
        _pack_body,
        grid=(P_GRID,),
        in_specs=[sub(0), sub(1), sub(2), sub(3),
                  sub(0), sub(1), sub(2), sub(3),
                  pl.BlockSpec((128, 128), lambda i: (0, 0))],
        out_specs=[pl.BlockSpec((SUB, 128), lambda i: (i, 0)),
                   pl.BlockSpec((SUB, 128), lambda i: (i, 0))],
        out_shape=[line_t, line_t],
    )(ut_t, ut_t, ut_t, ut_t, it_t, it_t, it_t, it_t, eye)
    return uq.reshape(QROWS * 4, D), iq.reshape(QROWS * 4, D)


def _gather_body(ut_hbm, it_hbm, ui_hbm, ii_hbm, u_hbm, v_hbm,
                 uidx_v, iidx_v, urows_v, irows_v, sem):
    wid = lax.axis_index("s") * NC + lax.axis_index("c")
    row0 = wid * N_CHUNK
    pltpu.sync_copy(ui_hbm.at[pl.ds(row0, N_CHUNK)], uidx_v)
    pltpu.sync_copy(ii_hbm.at[pl.ds(row0, N_CHUNK)], iidx_v)
    copies = []
    for j in range(N_CHUNK):
        copies.append(pltpu.async_copy(
            ut_hbm.at[uidx_v.at[j]], urows_v.at[pl.ds(j * CHUNK, CHUNK)], sem))
        copies.append(pltpu.async_copy(
            it_hbm.at[iidx_v.at[j]], irows_v.at[pl.ds(j * CHUNK, CHUNK)], sem))
    for c in copies:
        c.wait()
    base = wid * B_PER_W
    pltpu.sync_copy(urows_v, u_hbm.at[pl.ds(base, B_PER_W)])
    pltpu.sync_copy(irows_v, v_hbm.at[pl.ds(base, B_PER_W)])


def _sc_gather(user_table, item_table, user_idx, item_idx):
    mesh = plsc.VectorSubcoreMesh(core_axis_name="c", subcore_axis_name="s")
    rows_t = jax.ShapeDtypeStruct((BATCH, D), jnp.float32)
    k = pl.kernel(
        _gather_body,
        out_type=[rows_t, rows_t],
        mesh=mesh,
        compiler_params=pltpu.CompilerParams(use_tc_tiling_on_sc=False),
        scratch_types=[
            pltpu.VMEM((N_CHUNK, CHUNK), jnp.int32),
            pltpu.VMEM((N_CHUNK, CHUNK), jnp.int32),
            pltpu.VMEM((B_PER_W, D), jnp.float32),
            pltpu.VMEM((B_PER_W, D), jnp.float32),
            pltpu.SemaphoreType.DMA,
        ],
    )
    return k(user_table, item_table,
             user_idx.reshape(IDX_ROWS, CHUNK), item_idx.reshape(IDX_ROWS, CHUNK))


BLK = 2048


def _mlp_body(u_ref, v_ref, w1u_ref, w1v_ref, b1_ref, w2_ref, b2_ref,
              wo_ref, bo_ref, o_ref):
    h = u_ref[...] @ w1u_ref[...] + v_ref[...] @ w1v_ref[...] + b1_ref[...]
    h = jnp.maximum(h, 0.0)
    h = jnp.maximum(h @ w2_ref[...] + b2_ref[...], 0.0)
    o_ref[...] = h @ wo_ref[...] + bo_ref[...]


def _tc_mlp(u, v, W1, b1, W2, b2, Wo, bo):
    w1u, w1v = W1[:D], W1[D:]
    grid = (BATCH // BLK,)
    full = lambda shape: pl.BlockSpec(shape, lambda i: (0, 0))
    out = pl.pallas_call(
        _mlp_body,
        grid=grid,
        in_specs=[
            pl.BlockSpec((BLK, D), lambda i: (i, 0)),
            pl.BlockSpec((BLK, D), lambda i: (i, 0)),
            full((D, 64)),
            full((D, 64)),
            full((1, 64)),
            full((64, 32)),
            full((1, 32)),
            full((32, 1)),
            full((1, 1)),
        ],
        out_specs=pl.BlockSpec((BLK, 1), lambda i: (i, 0)),
        out_shape=jax.ShapeDtypeStruct((BATCH, 1), jnp.float32),
    )(u, v, w1u, w1v, b1.reshape(1, 64), W2, b2.reshape(1, 32),
      Wo, bo.reshape(1, 1))
    return out[:, 0]


def kernel(user_indices, item_indices, user_table, item_table,
           W1, b1, W2, b2, Wo, bo):
    uq, iq = _tc_pack(user_table.T, item_table.T)

    def remap(r):
        i = r // PW
        rem = r % PW
        return 4 * (SUB * i + rem % SUB) + rem // SUB

    uidx = remap(user_indices)
    iidx = remap(item_indices)
    u, v = _sc_gather(uq, iq, uidx, iidx)
    return _tc_mlp(u, v, W1, b1, W2, b2, Wo, bo)


# speedup vs baseline: 30.9751x; 1.3005x over previous
import functools

import jax
import jax.numpy as jnp
from jax import lax
from jax.experimental import pallas as pl
from jax.experimental.pallas import tpu as pltpu
from jax.experimental.pallas import tpu_sc as plsc

NC = 2
NS = 16
NW = NC * NS

BATCH = 16384
D = 32
N_ROWS = 1000000
PW = 8192
P_GRID = (N_ROWS + PW - 1) // PW
QROWS = P_GRID * (PW // 4)
B_PER_W = BATCH // NW
CHUNK = 128
N_CHUNK = B_PER_W // CHUNK
IDX_ROWS = BATCH // CHUNK

SUB = PW // 4


def _pack_body(x0, x1, x2, x3, y0, y1, y2, y3, eye_ref, uo_ref, io_ref):
    e = eye_ref[...]
    xs = jnp.concatenate([x0[...], x1[...], x2[...], x3[...]], axis=0)
    ys = jnp.concatenate([y0[...], y1[...], y2[...], y3[...]], axis=0)
    uo_ref[...] = lax.dot_general(
        xs, e, (((0,), (0,)), ((), ())), preferred_element_type=jnp.float32)
    io_ref[...] = lax.dot_general(
        ys, e, (((0,), (0,)), ((), ())), preferred_element_type=jnp.float32)


def _tc_pack(ut_t, it_t):
    eye = jnp.eye(128, dtype=jnp.float32)
    line_t = jax.ShapeDtypeStruct((QROWS, 128), jnp.float32)
    last = N_ROWS // SUB
    sub = lambda g: pl.BlockSpec(
        (D, SUB), lambda i, g=g: (0, jnp.minimum(4 * i + g, last)))
    uq, iq = pl.pallas_call(
        _pack_body,
        grid=(P_GRID,),
        in_specs=[sub(0), sub(1), sub(2), sub(3),
                  sub(0), sub(1), sub(2), sub(3),
                  pl.BlockSpec((128, 128), lambda i: (0, 0))],
        out_specs=[pl.BlockSpec((SUB, 128), lambda i: (i, 0)),
                   pl.BlockSpec((SUB, 128), lambda i: (i, 0))],
        out_shape=[line_t, line_t],
    )(ut_t, ut_t, ut_t, ut_t, it_t, it_t, it_t, it_t, eye)
    return uq.reshape(QROWS * 4, D), iq.reshape(QROWS * 4, D)


def _gather_body(ut_hbm, it_hbm, ui_hbm, ii_hbm, u_hbm, v_hbm,
                 uidx_v, iidx_v, urows_v, irows_v, sem):
    wid = lax.axis_index("s") * NC + lax.axis_index("c")
    row0 = wid * N_CHUNK
    pltpu.sync_copy(ui_hbm.at[pl.ds(row0, N_CHUNK)], uidx_v)
    pltpu.sync_copy(ii_hbm.at[pl.ds(row0, N_CHUNK)], iidx_v)
    copies = []
    for j in range(N_CHUNK):
        copies.append(pltpu.async_copy(
            ut_hbm.at[uidx_v.at[j]], urows_v.at[pl.ds(j * CHUNK, CHUNK)], sem))
        copies.append(pltpu.async_copy(
            it_hbm.at[iidx_v.at[j]], irows_v.at[pl.ds(j * CHUNK, CHUNK)], sem))
    for c in copies:
        c.wait()
    base = wid * B_PER_W
    pltpu.sync_copy(urows_v, u_hbm.at[pl.ds(base, B_PER_W)])
    pltpu.sync_copy(irows_v, v_hbm.at[pl.ds(base, B_PER_W)])


def _sc_gather(user_table, item_table, user_idx, item_idx):
    mesh = plsc.VectorSubcoreMesh(core_axis_name="c", subcore_axis_name="s")
    rows_t = jax.ShapeDtypeStruct((BATCH, D), jnp.float32)
    k = pl.kernel(
        _gather_body,
        out_type=[rows_t, rows_t],
        mesh=mesh,
        compiler_params=pltpu.CompilerParams(use_tc_tiling_on_sc=False),
        scratch_types=[
            pltpu.VMEM((N_CHUNK, CHUNK), jnp.int32),
            pltpu.VMEM((N_CHUNK, CHUNK), jnp.int32),
            pltpu.VMEM((B_PER_W, D), jnp.float32),
            pltpu.VMEM((B_PER_W, D), jnp.float32),
            pltpu.SemaphoreType.DMA,
        ],
    )
    return k(user_table, item_table,
             user_idx.reshape(IDX_ROWS, CHUNK), item_idx.reshape(IDX_ROWS, CHUNK))


BLK = 2048


def _mlp_body(u_ref, v_ref, w1u_ref, w1v_ref, b1_ref, w2_ref, b2_ref,
              wo_ref, bo_ref, o_ref):
    h = u_ref[...] @ w1u_ref[...] + v_ref[...] @ w1v_ref[...] + b1_ref[...]
    h = jnp.maximum(h, 0.0)
    h = jnp.maximum(h @ w2_ref[...] + b2_ref[...], 0.0)
    o_ref[...] = h @ wo_ref[...] + bo_ref[...]


def _tc_mlp(u, v, W1, b1, W2, b2, Wo, bo):
    w1u, w1v = W1[:D], W1[D:]
    grid = (BATCH // BLK,)
    full = lambda shape: pl.BlockSpec(shape, lambda i: (0, 0))
    out = pl.pallas_call(
        _mlp_body,
        grid=grid,
        in_specs=[
            pl.BlockSpec((BLK, D), lambda i: (i, 0)),
            pl.BlockSpec((BLK, D), lambda i: (i, 0)),
            full((D, 64)),
            full((D, 64)),
            full((1, 64)),
            full((64, 32)),
            full((1, 32)),
            full((32, 1)),
            full((1, 1)),
        ],
        out_specs=pl.BlockSpec((BLK, 1), lambda i: (i, 0)),
        out_shape=jax.ShapeDtypeStruct((BATCH, 1), jnp.float32),
    )(u, v, w1u, w1v, b1.reshape(1, 64), W2, b2.reshape(1, 32),
      Wo, bo.reshape(1, 1))
    return out[:, 0]


def kernel(user_indices, item_indices, user_table, item_table,
           W1, b1, W2, b2, Wo, bo):
    uq, iq = _tc_pack(user_table.T, item_table.T)

    def remap(r):
        i = r // PW
        rem = r % PW
        return 4 * (SUB * i + rem % SUB) + rem // SUB

    uidx = remap(user_indices)
    iidx = remap(item_indices)
    u, v = _sc_gather(uq, iq, uidx, iidx)
    return _tc_mlp(u, v, W1, b1, W2, b2, Wo, bo)


# speedup vs baseline: 35.3200x; 1.1403x over previous
import functools

import jax
import jax.numpy as jnp
from jax import lax
from jax.experimental import pallas as pl
from jax.experimental.pallas import tpu as pltpu
from jax.experimental.pallas import tpu_sc as plsc

NC = 2
NS = 16
NW = NC * NS

BATCH = 16384
D = 32
N_ROWS = 1000000
PW = 16384
P_GRID = (N_ROWS + PW - 1) // PW
QROWS = P_GRID * (PW // 4)
B_PER_W = BATCH // NW
CHUNK = 128
N_CHUNK = B_PER_W // CHUNK
IDX_ROWS = BATCH // CHUNK

SUB = PW // 4


def _pack_body(x0, x1, x2, x3, y0, y1, y2, y3, eye_ref, uo_ref, io_ref):
    e = eye_ref[...]
    xs = jnp.concatenate([x0[...], x1[...], x2[...], x3[...]], axis=0)
    ys = jnp.concatenate([y0[...], y1[...], y2[...], y3[...]], axis=0)
    uo_ref[...] = lax.dot_general(
        xs, e, (((0,), (0,)), ((), ())), preferred_element_type=jnp.float32)
    io_ref[...] = lax.dot_general(
        ys, e, (((0,), (0,)), ((), ())), preferred_element_type=jnp.float32)


def _tc_pack(ut_t, it_t):
    eye = jnp.eye(128, dtype=jnp.float32)
    line_t = jax.ShapeDtypeStruct((QROWS, 128), jnp.float32)
    last = N_ROWS // SUB
    sub = lambda g: pl.BlockSpec(
        (D, SUB), lambda i, g=g: (0, jnp.minimum(4 * i + g, last)))
    uq, iq = pl.pallas_call(
        _pack_body,
        grid=(P_GRID,),
        in_specs=[sub(0), sub(1), sub(2), sub(3),
                  sub(0), sub(1), sub(2), sub(3),
                  pl.BlockSpec((128, 128), lambda i: (0, 0))],
        out_specs=[pl.BlockSpec((SUB, 128), lambda i: (i, 0)),
                   pl.BlockSpec((SUB, 128), lambda i: (i, 0))],
        out_shape=[line_t, line_t],
    )(ut_t, ut_t, ut_t, ut_t, it_t, it_t, it_t, it_t, eye)
    return uq.reshape(QROWS * 4, D), iq.reshape(QROWS * 4, D)


def _gather_body(ut_hbm, it_hbm, ui_hbm, ii_hbm, u_hbm, v_hbm,
                 uidx_v, iidx_v, urows_v, irows_v, sem):
    wid = lax.axis_index("s") * NC + lax.axis_index("c")
    row0 = wid * N_CHUNK
    pltpu.sync_copy(ui_hbm.at[pl.ds(row0, N_CHUNK)], uidx_v)
    pltpu.sync_copy(ii_hbm.at[pl.ds(row0, N_CHUNK)], iidx_v)
    copies = []
    for j in range(N_CHUNK):
        copies.append(pltpu.async_copy(
            ut_hbm.at[uidx_v.at[j]], urows_v.at[pl.ds(j * CHUNK, CHUNK)], sem))
        copies.append(pltpu.async_copy(
            it_hbm.at[iidx_v.at[j]], irows_v.at[pl.ds(j * CHUNK, CHUNK)], sem))
    for c in copies:
        c.wait()
    base = wid * B_PER_W
    pltpu.sync_copy(urows_v, u_hbm.at[pl.ds(base, B_PER_W)])
    pltpu.sync_copy(irows_v, v_hbm.at[pl.ds(base, B_PER_W)])


def _sc_gather(user_table, item_table, user_idx, item_idx):
    mesh = plsc.VectorSubcoreMesh(core_axis_name="c", subcore_axis_name="s")
    rows_t = jax.ShapeDtypeStruct((BATCH, D), jnp.float32)
    k = pl.kernel(
        _gather_body,
        out_type=[rows_t, rows_t],
        mesh=mesh,
        compiler_params=pltpu.CompilerParams(use_tc_tiling_on_sc=False),
        scratch_types=[
            pltpu.VMEM((N_CHUNK, CHUNK), jnp.int32),
            pltpu.VMEM((N_CHUNK, CHUNK), jnp.int32),
            pltpu.VMEM((B_PER_W, D), jnp.float32),
            pltpu.VMEM((B_PER_W, D), jnp.float32),
            pltpu.SemaphoreType.DMA,
        ],
    )
    return k(user_table, item_table,
             user_idx.reshape(IDX_ROWS, CHUNK), item_idx.reshape(IDX_ROWS, CHUNK))


BLK = 2048


def _mlp_body(u_ref, v_ref, w1u_ref, w1v_ref, b1_ref, w2_ref, b2_ref,
              wo_ref, bo_ref, o_ref):
    h = u_ref[...] @ w1u_ref[...] + v_ref[...] @ w1v_ref[...] + b1_ref[...]
    h = jnp.maximum(h, 0.0)
    h = jnp.maximum(h @ w2_ref[...] + b2_ref[...], 0.0)
    o_ref[...] = h @ wo_ref[...] + bo_ref[...]


def _tc_mlp(u, v, W1, b1, W2, b2, Wo, bo):
    w1u, w1v = W1[:D], W1[D:]
    grid = (BATCH // BLK,)
    full = lambda shape: pl.BlockSpec(shape, lambda i: (0, 0))
    out = pl.pallas_call(
        _mlp_body,
        grid=grid,
        in_specs=[
            pl.BlockSpec((BLK, D), lambda i: (i, 0)),
            pl.BlockSpec((BLK, D), lambda i: (i, 0)),
            full((D, 64)),
            full((D, 64)),
            full((1, 64)),
            full((64, 32)),
            full((1, 32)),
            full((32, 1)),
            full((1, 1)),
        ],
        out_specs=pl.BlockSpec((BLK, 1), lambda i: (i, 0)),
        out_shape=jax.ShapeDtypeStruct((BATCH, 1), jnp.float32),
    )(u, v, w1u, w1v, b1.reshape(1, 64), W2, b2.reshape(1, 32),
      Wo, bo.reshape(1, 1))
    return out[:, 0]


def kernel(user_indices, item_indices, user_table, item_table,
           W1, b1, W2, b2, Wo, bo):
    uq, iq = _tc_pack(user_table.T, item_table.T)

    def remap(r):
        i = r // PW
        rem = r % PW
        return 4 * (SUB * i + rem % SUB) + rem // SUB

    uidx = remap(user_indices)
    iidx = remap(item_indices)
    u, v = _sc_gather(uq, iq, uidx, iidx)
    return _tc_mlp(u, v, W1, b1, W2, b2, Wo, bo)


# speedup vs baseline: 36.1204x; 1.0227x over previous
import functools

import jax
import jax.numpy as jnp
from jax import lax
from jax.experimental import pallas as pl
from jax.experimental.pallas import tpu as pltpu
from jax.experimental.pallas import tpu_sc as plsc

NC = 2
NS = 16
NW = NC * NS

BATCH = 16384
D = 32
N_ROWS = 1000000
PW = 32768
P_GRID = (N_ROWS + PW - 1) // PW
QROWS = P_GRID * (PW // 4)
B_PER_W = BATCH // NW
CHUNK = 128
N_CHUNK = B_PER_W // CHUNK
IDX_ROWS = BATCH // CHUNK

SUB = PW // 4


def _pack_body(x0, x1, x2, x3, y0, y1, y2, y3, eye_ref, uo_ref, io_ref):
    e = eye_ref[...]
    xs = jnp.concatenate([x0[...], x1[...], x2[...], x3[...]], axis=0)
    ys = jnp.concatenate([y0[...], y1[...], y2[...], y3[...]], axis=0)
    uo_ref[...] = lax.dot_general(
        xs, e, (((0,), (0,)), ((), ())), preferred_element_type=jnp.float32)
    io_ref[...] = lax.dot_general(
        ys, e, (((0,), (0,)), ((), ())), preferred_element_type=jnp.float32)


def _tc_pack(ut_t, it_t):
    eye = jnp.eye(128, dtype=jnp.float32)
    line_t = jax.ShapeDtypeStruct((QROWS, 128), jnp.float32)
    last = N_ROWS // SUB
    sub = lambda g: pl.BlockSpec(
        (D, SUB), lambda i, g=g: (0, jnp.minimum(4 * i + g, last)))
    uq, iq = pl.pallas_call(
        _pack_body,
        grid=(P_GRID,),
        in_specs=[sub(0), sub(1), sub(2), sub(3),
                  sub(0), sub(1), sub(2), sub(3),
                  pl.BlockSpec((128, 128), lambda i: (0, 0))],
        out_specs=[pl.BlockSpec((SUB, 128), lambda i: (i, 0)),
                   pl.BlockSpec((SUB, 128), lambda i: (i, 0))],
        out_shape=[line_t, line_t],
    )(ut_t, ut_t, ut_t, ut_t, it_t, it_t, it_t, it_t, eye)
    return uq.reshape(QROWS * 4, D), iq.reshape(QROWS * 4, D)


def _gather_body(ut_hbm, it_hbm, ui_hbm, ii_hbm, u_hbm, v_hbm,
                 uidx_v, iidx_v, urows_v, irows_v, sem):
    wid = lax.axis_index("s") * NC + lax.axis_index("c")
    row0 = wid * N_CHUNK
    pltpu.sync_copy(ui_hbm.at[pl.ds(row0, N_CHUNK)], uidx_v)
    pltpu.sync_copy(ii_hbm.at[pl.ds(row0, N_CHUNK)], iidx_v)
    copies = []
    for j in range(N_CHUNK):
        copies.append(pltpu.async_copy(
            ut_hbm.at[uidx_v.at[j]], urows_v.at[pl.ds(j * CHUNK, CHUNK)], sem))
        copies.append(pltpu.async_copy(
            it_hbm.at[iidx_v.at[j]], irows_v.at[pl.ds(j * CHUNK, CHUNK)], sem))
    for c in copies:
        c.wait()
    base = wid * B_PER_W
    pltpu.sync_copy(urows_v, u_hbm.at[pl.ds(base, B_PER_W)])
    pltpu.sync_copy(irows_v, v_hbm.at[pl.ds(base, B_PER_W)])


def _sc_gather(user_table, item_table, user_idx, item_idx):
    mesh = plsc.VectorSubcoreMesh(core_axis_name="c", subcore_axis_name="s")
    rows_t = jax.ShapeDtypeStruct((BATCH, D), jnp.float32)
    k = pl.kernel(
        _gather_body,
        out_type=[rows_t, rows_t],
        mesh=mesh,
        compiler_params=pltpu.CompilerParams(use_tc_tiling_on_sc=False),
        scratch_types=[
            pltpu.VMEM((N_CHUNK, CHUNK), jnp.int32),
            pltpu.VMEM((N_CHUNK, CHUNK), jnp.int32),
            pltpu.VMEM((B_PER_W, D), jnp.float32),
            pltpu.VMEM((B_PER_W, D), jnp.float32),
            pltpu.SemaphoreType.DMA,
        ],
    )
    return k(user_table, item_table,
             user_idx.reshape(IDX_ROWS, CHUNK), item_idx.reshape(IDX_ROWS, CHUNK))


BLK = 2048


def _mlp_body(u_ref, v_ref, w1u_ref, w1v_ref, b1_ref, w2_ref, b2_ref,
              wo_ref, bo_ref, o_ref):
    h = u_ref[...] @ w1u_ref[...] + v_ref[...] @ w1v_ref[...] + b1_ref[...]
    h = jnp.maximum(h, 0.0)
    h = jnp.maximum(h @ w2_ref[...] + b2_ref[...], 0.0)
    o_ref[...] = h @ wo_ref[...] + bo_ref[...]


def _tc_mlp(u, v, W1, b1, W2, b2, Wo, bo):
    w1u, w1v = W1[:D], W1[D:]
    grid = (BATCH // BLK,)
    full = lambda shape: pl.BlockSpec(shape, lambda i: (0, 0))
    out = pl.pallas_call(
        _mlp_body,
        grid=grid,
        in_specs=[
            pl.BlockSpec((BLK, D), lambda i: (i, 0)),
            pl.BlockSpec((BLK, D), lambda i: (i, 0)),
            full((D, 64)),
            full((D, 64)),
            full((1, 64)),
            full((64, 32)),
            full((1, 32)),
            full((32, 1)),
            full((1, 1)),
        ],
        out_specs=pl.BlockSpec((BLK, 1), lambda i: (i, 0)),
        out_shape=jax.ShapeDtypeStruct((BATCH, 1), jnp.float32),
    )(u, v, w1u, w1v, b1.reshape(1, 64), W2, b2.reshape(1, 32),
      Wo, bo.reshape(1, 1))
    return out[:, 0]


def kernel(user_indices, item_indices, user_table, item_table,
           W1, b1, W2, b2, Wo, bo):
    uq, iq = _tc_pack(user_table.T, item_table.T)

    def remap(r):
        i = r // PW
        rem = r % PW
        return 4 * (SUB * i + rem % SUB) + rem // SUB

    uidx = remap(user_indices)
    iidx = remap(item_indices)
    u, v = _sc_gather(uq, iq, uidx, iidx)
    return _tc_mlp(u, v, W1, b1, W2, b2, Wo, bo)


# speedup vs baseline: 36.3216x; 1.0056x over previous
import functools

import jax
import jax.numpy as jnp
from jax import lax
from jax.experimental import pallas as pl
from jax.experimental.pallas import tpu as pltpu
from jax.experimental.pallas import tpu_sc as plsc

NC = 2
NS = 16
NW = NC * NS

BATCH = 16384
D = 32
N_ROWS = 1000000
PW = 32768
P_GRID = (N_ROWS + PW - 1) // PW
QROWS = P_GRID * (PW // 4)
B_PER_W = BATCH // NW
CHUNK = 128
N_CHUNK = B_PER_W // CHUNK
IDX_ROWS = BATCH // CHUNK

SUB = PW // 4


def _pack_body(x0, x1, x2, x3, y0, y1, y2, y3, eye_ref, uo_ref, io_ref):
    e = eye_ref[...]
    xs = jnp.concatenate([x0[...], x1[...], x2[...], x3[...]],
                         axis=0).astype(jnp.bfloat16)
    ys = jnp.concatenate([y0[...], y1[...], y2[...], y3[...]],
                         axis=0).astype(jnp.bfloat16)
    uo_ref[...] = lax.dot_general(
        xs, e, (((0,), (0,)), ((), ())), preferred_element_type=jnp.float32)
    io_ref[...] = lax.dot_general(
        ys, e, (((0,), (0,)), ((), ())), preferred_element_type=jnp.float32)


def _tc_pack(ut_t, it_t):
    eye = jnp.eye(128, dtype=jnp.bfloat16)
    line_t = jax.ShapeDtypeStruct((QROWS, 128), jnp.float32)
    last = N_ROWS // SUB
    sub = lambda g: pl.BlockSpec(
        (D, SUB), lambda i, g=g: (0, jnp.minimum(4 * i + g, last)))
    uq, iq = pl.pallas_call(
        _pack_body,
        grid=(P_GRID,),
        in_specs=[sub(0), sub(1), sub(2), sub(3),
                  sub(0), sub(1), sub(2), sub(3),
                  pl.BlockSpec((128, 128), lambda i: (0, 0))],
        out_specs=[pl.BlockSpec((SUB, 128), lambda i: (i, 0)),
                   pl.BlockSpec((SUB, 128), lambda i: (i, 0))],
        out_shape=[line_t, line_t],
    )(ut_t, ut_t, ut_t, ut_t, it_t, it_t, it_t, it_t, eye)
    return uq.reshape(QROWS * 4, D), iq.reshape(QROWS * 4, D)


def _gather_body(ut_hbm, it_hbm, ui_hbm, ii_hbm, u_hbm, v_hbm,
                 uidx_v, iidx_v, urows_v, irows_v, sem):
    wid = lax.axis_index("s") * NC + lax.axis_index("c")
    row0 = wid * N_CHUNK
    pltpu.sync_copy(ui_hbm.at[pl.ds(row0, N_CHUNK)], uidx_v)
    pltpu.sync_copy(ii_hbm.at[pl.ds(row0, N_CHUNK)], iidx_v)
    copies = []
    for j in range(N_CHUNK):
        copies.append(pltpu.async_copy(
            ut_hbm.at[uidx_v.at[j]], urows_v.at[pl.ds(j * CHUNK, CHUNK)], sem))
        copies.append(pltpu.async_copy(
            it_hbm.at[iidx_v.at[j]], irows_v.at[pl.ds(j * CHUNK, CHUNK)], sem))
    for c in copies:
        c.wait()
    base = wid * B_PER_W
    pltpu.sync_copy(urows_v, u_hbm.at[pl.ds(base, B_PER_W)])
    pltpu.sync_copy(irows_v, v_hbm.at[pl.ds(base, B_PER_W)])


def _sc_gather(user_table, item_table, user_idx, item_idx):
    mesh = plsc.VectorSubcoreMesh(core_axis_name="c", subcore_axis_name="s")
    rows_t = jax.ShapeDtypeStruct((BATCH, D), jnp.float32)
    k = pl.kernel(
        _gather_body,
        out_type=[rows_t, rows_t],
        mesh=mesh,
        compiler_params=pltpu.CompilerParams(use_tc_tiling_on_sc=False),
        scratch_types=[
            pltpu.VMEM((N_CHUNK, CHUNK), jnp.int32),
            pltpu.VMEM((N_CHUNK, CHUNK), jnp.int32),
            pltpu.VMEM((B_PER_W, D), jnp.float32),
            pltpu.VMEM((B_PER_W, D), jnp.float32),
            pltpu.SemaphoreType.DMA,
        ],
    )
    return k(user_table, item_table,
             user_idx.reshape(IDX_ROWS, CHUNK), item_idx.reshape(IDX_ROWS, CHUNK))


BLK = 2048


def _mlp_body(u_ref, v_ref, w1u_ref, w1v_ref, b1_ref, w2_ref, b2_ref,
              wo_ref, bo_ref, o_ref):
    h = u_ref[...] @ w1u_ref[...] + v_ref[...] @ w1v_ref[...] + b1_ref[...]
    h = jnp.maximum(h, 0.0)
    h = jnp.maximum(h @ w2_ref[...] + b2_ref[...], 0.0)
    o_ref[...] = h @ wo_ref[...] + bo_ref[...]


def _tc_mlp(u, v, W1, b1, W2, b2, Wo, bo):
    w1u, w1v = W1[:D], W1[D:]
    grid = (BATCH // BLK,)
    full = lambda shape: pl.BlockSpec(shape, lambda i: (0, 0))
    out = pl.pallas_call(
        _mlp_body,
        grid=grid,
        in_specs=[
            pl.BlockSpec((BLK, D), lambda i: (i, 0)),
            pl.BlockSpec((BLK, D), lambda i: (i, 0)),
            full((D, 64)),
            full((D, 64)),
            full((1, 64)),
            full((64, 32)),
            full((1, 32)),
            full((32, 1)),
            full((1, 1)),
        ],
        out_specs=pl.BlockSpec((BLK, 1), lambda i: (i, 0)),
        out_shape=jax.ShapeDtypeStruct((BATCH, 1), jnp.float32),
    )(u, v, w1u, w1v, b1.reshape(1, 64), W2, b2.reshape(1, 32),
      Wo, bo.reshape(1, 1))
    return out[:, 0]


def kernel(user_indices, item_indices, user_table, item_table,
           W1, b1, W2, b2, Wo, bo):
    uq, iq = _tc_pack(user_table.T, item_table.T)

    def remap(r):
        i = r // PW
        rem = r % PW
        return 4 * (SUB * i + rem % SUB) + rem // SUB

    uidx = remap(user_indices)
    iidx = remap(item_indices)
    u, v = _sc_gather(uq, iq, uidx, iidx)
    return _tc_mlp(u, v, W1, b1, W2, b2, Wo, bo)
